# baseline (speedup 1.0000x reference)
import functools

import jax
import jax.numpy as jnp
from jax import lax
from jax.experimental import pallas as pl
from jax.experimental.pallas import tpu as pltpu
from jax.experimental.pallas import tpu_sc as plsc

BATCH = 16384
D_EMB = 128
N_SPLIT = 4
CHUNK_B = BATCH // N_SPLIT

_NUM_CORES = 2
_NUM_SUBCORES = 16
_NUM_WORKERS = _NUM_CORES * _NUM_SUBCORES
_BPW = CHUNK_B // _NUM_WORKERS
_CHUNK = 128
_NCHUNK = _BPW // _CHUNK


@functools.cache
def _make_sc_gather(chunk_idx):
    mesh = plsc.VectorSubcoreMesh(core_axis_name="c", subcore_axis_name="s")

    @functools.partial(
        pl.kernel,
        mesh=mesh,
        out_type=(
            jax.ShapeDtypeStruct((CHUNK_B, D_EMB), jnp.float32),
            jax.ShapeDtypeStruct((CHUNK_B, D_EMB), jnp.float32),
        ),
        scratch_types=[
            pltpu.VMEM((_BPW,), jnp.int32),
            pltpu.VMEM((_BPW,), jnp.int32),
            pltpu.VMEM((_BPW, D_EMB), jnp.float32),
            pltpu.SemaphoreType.DMA,
        ],
    )
    def _sc_gather(users_hbm, movies_hbm, ut_hbm, mt_hbm, u_out, m_out,
                   uidx_v, midx_v, rows_v, sem):
        wid = lax.axis_index("s") * _NUM_CORES + lax.axis_index("c")
        base = wid * _BPW
        src_base = chunk_idx * CHUNK_B + wid * _BPW
        pltpu.sync_copy(users_hbm.at[pl.ds(src_base, _BPW)], uidx_v)
        pltpu.sync_copy(movies_hbm.at[pl.ds(src_base, _BPW)], midx_v)

        def gather_table(table_hbm, idx_v, out_hbm):
            copies = []
            for j in range(_NCHUNK):
                cp = pltpu.async_copy(
                    table_hbm.at[idx_v.at[pl.ds(j * _CHUNK, _CHUNK)]],
                    rows_v.at[pl.ds(j * _CHUNK, _CHUNK)],
                    sem,
                )
                copies.append(cp)
            for cp in copies:
                cp.wait()
            pltpu.sync_copy(rows_v, out_hbm.at[pl.ds(base, _BPW)])

        gather_table(ut_hbm, uidx_v, u_out)
        gather_table(mt_hbm, midx_v, m_out)

    return _sc_gather


_BB = 2048


def _mlp_body(u_ref, m_ref, w0_ref, b0_ref, w1_ref, b1_ref,
              wout_ref, bout_ref, o_ref):
    bf = jnp.bfloat16
    u = u_ref[...].astype(bf)
    m = m_ref[...].astype(bf)
    h0 = jnp.dot(u, w0_ref[0:D_EMB, :], preferred_element_type=jnp.float32)
    h0 += jnp.dot(m, w0_ref[D_EMB:2 * D_EMB, :],
                  preferred_element_type=jnp.float32)
    h0 = jnp.maximum(h0 + b0_ref[...], 0.0).astype(bf)
    h1 = jnp.dot(h0, w1_ref[...], preferred_element_type=jnp.float32)
    h1 = jnp.maximum(h1 + b1_ref[...], 0.0).astype(bf)
    out = jnp.dot(h1, wout_ref[...], preferred_element_type=jnp.float32)
    o_ref[...] = out + bout_ref[...]


def _mlp(u_emb, m_emb, W0, b0, W1, b1, Wout, bout):
    h0_dim = W0.shape[1]
    h1_dim = W1.shape[1]
    grid = (CHUNK_B // _BB,)
    return pl.pallas_call(
        _mlp_body,
        grid=grid,
        in_specs=[
            pl.BlockSpec((_BB, D_EMB), lambda i: (i, 0)),
            pl.BlockSpec((_BB, D_EMB), lambda i: (i, 0)),
            pl.BlockSpec((2 * D_EMB, h0_dim), lambda i: (0, 0)),
            pl.BlockSpec((1, h0_dim), lambda i: (0, 0)),
            pl.BlockSpec((h0_dim, h1_dim), lambda i: (0, 0)),
            pl.BlockSpec((1, h1_dim), lambda i: (0, 0)),
            pl.BlockSpec((h1_dim, 1), lambda i: (0, 0)),
            pl.BlockSpec((1, 1), lambda i: (0, 0)),
        ],
        out_specs=pl.BlockSpec((_BB, 1), lambda i: (i, 0)),
        out_shape=jax.ShapeDtypeStruct((CHUNK_B, 1), jnp.float32),
        compiler_params=pltpu.CompilerParams(
            dimension_semantics=("arbitrary",),
        ),
    )(u_emb, m_emb, W0, b0, W1, b1, Wout, bout)


def kernel(users, movies, user_table, movie_table, W0, b0, W1, b1, Wout, bout):
    users = users.astype(jnp.int32)
    movies = movies.astype(jnp.int32)
    W0 = W0.astype(jnp.bfloat16)
    W1 = W1.astype(jnp.bfloat16)
    Wout = Wout.astype(jnp.bfloat16)
    b0 = b0.reshape(1, -1)
    b1 = b1.reshape(1, -1)
    bout = bout.reshape(1, 1)
    outs = []
    for k in range(N_SPLIT):
        u_emb, m_emb = _make_sc_gather(k)(users, movies,
                                          user_table, movie_table)
        outs.append(_mlp(u_emb, m_emb, W0, b0, W1, b1, Wout, bout))
    return jnp.concatenate(outs, axis=0)


# speedup vs baseline: 1.1421x; 1.1421x over previous
import functools

import jax
import jax.numpy as jnp
from jax import lax
from jax.experimental import pallas as pl
from jax.experimental.pallas import tpu as pltpu
from jax.experimental.pallas import tpu_sc as plsc

BATCH = 16384
D_EMB = 128
N_SPLIT = 2
CHUNK_B = BATCH // N_SPLIT

_NUM_CORES = 2
_NUM_SUBCORES = 16
_NUM_WORKERS = _NUM_CORES * _NUM_SUBCORES
_BPW = CHUNK_B // _NUM_WORKERS
_CHUNK = 128
_NCHUNK = _BPW // _CHUNK


@functools.cache
def _make_sc_gather(chunk_idx):
    mesh = plsc.VectorSubcoreMesh(core_axis_name="c", subcore_axis_name="s")

    @functools.partial(
        pl.kernel,
        mesh=mesh,
        out_type=(
            jax.ShapeDtypeStruct((CHUNK_B, D_EMB), jnp.float32),
            jax.ShapeDtypeStruct((CHUNK_B, D_EMB), jnp.float32),
        ),
        scratch_types=[
            pltpu.VMEM((_BPW,), jnp.int32),
            pltpu.VMEM((_BPW,), jnp.int32),
            pltpu.VMEM((_BPW, D_EMB), jnp.float32),
            pltpu.SemaphoreType.DMA,
        ],
    )
    def _sc_gather(users_hbm, movies_hbm, ut_hbm, mt_hbm, u_out, m_out,
                   uidx_v, midx_v, rows_v, sem):
        wid = lax.axis_index("s") * _NUM_CORES + lax.axis_index("c")
        base = wid * _BPW
        src_base = chunk_idx * CHUNK_B + wid * _BPW
        pltpu.sync_copy(users_hbm.at[pl.ds(src_base, _BPW)], uidx_v)
        pltpu.sync_copy(movies_hbm.at[pl.ds(src_base, _BPW)], midx_v)

        def gather_table(table_hbm, idx_v, out_hbm):
            copies = []
            for j in range(_NCHUNK):
                cp = pltpu.async_copy(
                    table_hbm.at[idx_v.at[pl.ds(j * _CHUNK, _CHUNK)]],
                    rows_v.at[pl.ds(j * _CHUNK, _CHUNK)],
                    sem,
                )
                copies.append(cp)
            for cp in copies:
                cp.wait()
            pltpu.sync_copy(rows_v, out_hbm.at[pl.ds(base, _BPW)])

        gather_table(ut_hbm, uidx_v, u_out)
        gather_table(mt_hbm, midx_v, m_out)

    return _sc_gather


_BB = 2048


def _mlp_body(u_ref, m_ref, w0_ref, b0_ref, w1_ref, b1_ref,
              wout_ref, bout_ref, o_ref):
    bf = jnp.bfloat16
    u = u_ref[...].astype(bf)
    m = m_ref[...].astype(bf)
    h0 = jnp.dot(u, w0_ref[0:D_EMB, :], preferred_element_type=jnp.float32)
    h0 += jnp.dot(m, w0_ref[D_EMB:2 * D_EMB, :],
                  preferred_element_type=jnp.float32)
    h0 = jnp.maximum(h0 + b0_ref[...], 0.0).astype(bf)
    h1 = jnp.dot(h0, w1_ref[...], preferred_element_type=jnp.float32)
    h1 = jnp.maximum(h1 + b1_ref[...], 0.0).astype(bf)
    out = jnp.dot(h1, wout_ref[...], preferred_element_type=jnp.float32)
    out = out + bout_ref[...]
    o_ref[...] = out.reshape(1, out.shape[0])


def _mlp(u_emb, m_emb, W0, b0, W1, b1, Wout, bout):
    h0_dim = W0.shape[1]
    h1_dim = W1.shape[1]
    grid = (CHUNK_B // _BB,)
    return pl.pallas_call(
        _mlp_body,
        grid=grid,
        in_specs=[
            pl.BlockSpec((_BB, D_EMB), lambda i: (i, 0)),
            pl.BlockSpec((_BB, D_EMB), lambda i: (i, 0)),
            pl.BlockSpec((2 * D_EMB, h0_dim), lambda i: (0, 0)),
            pl.BlockSpec((1, h0_dim), lambda i: (0, 0)),
            pl.BlockSpec((h0_dim, h1_dim), lambda i: (0, 0)),
            pl.BlockSpec((1, h1_dim), lambda i: (0, 0)),
            pl.BlockSpec((h1_dim, 1), lambda i: (0, 0)),
            pl.BlockSpec((1, 1), lambda i: (0, 0)),
        ],
        out_specs=pl.BlockSpec((1, _BB), lambda i: (0, i)),
        out_shape=jax.ShapeDtypeStruct((1, CHUNK_B), jnp.float32),
        compiler_params=pltpu.CompilerParams(
            dimension_semantics=("arbitrary",),
        ),
    )(u_emb, m_emb, W0, b0, W1, b1, Wout, bout)


def kernel(users, movies, user_table, movie_table, W0, b0, W1, b1, Wout, bout):
    users = users.astype(jnp.int32)
    movies = movies.astype(jnp.int32)
    W0 = W0.astype(jnp.bfloat16)
    W1 = W1.astype(jnp.bfloat16)
    Wout = Wout.astype(jnp.bfloat16)
    b0 = b0.reshape(1, -1)
    b1 = b1.reshape(1, -1)
    bout = bout.reshape(1, 1)
    outs = []
    for k in range(N_SPLIT):
        u_emb, m_emb = _make_sc_gather(k)(users, movies,
                                          user_table, movie_table)
        outs.append(_mlp(u_emb, m_emb, W0, b0, W1, b1, Wout, bout))
    return jnp.concatenate(outs, axis=1).reshape(BATCH, 1)
